# baseline (speedup 1.0000x reference)
import jax
import jax.numpy as jnp
from jax.experimental import pallas as pl
from jax.experimental.pallas import tpu as pltpu

_SCALE = 0.044194173824159216
_TEMPERATURE = 0.2
_C = -_SCALE / _TEMPERATURE

_BC = 1024
_KDEP = 8
_ODEP = 8


def _k_copy(k_hbm, kbuf, ksem, g):
    return pltpu.make_async_copy(
        k_hbm.at[pl.ds(g * _BC, _BC), :],
        kbuf.at[g % _KDEP],
        ksem.at[g % _KDEP],
    )


def _o_copy(stag, o_hbm, osem, g):
    return pltpu.make_async_copy(
        stag.at[g % _ODEP],
        o_hbm.at[:, pl.ds(g * _BC, _BC)],
        osem.at[g % _ODEP],
    )


def _dist_kernel(q_ref, k_hbm, o_hbm, qs_ref, kbuf, stag, ksem, osem):
    n_chunks = k_hbm.shape[0] // _BC

    for g in range(min(_KDEP, n_chunks)):
        _k_copy(k_hbm, kbuf, ksem, g).start()

    qs_ref[...] = (q_ref[...] * _C).astype(jnp.bfloat16)
    qs = qs_ref[...]

    for g in range(n_chunks):
        _k_copy(k_hbm, kbuf, ksem, g).wait()
        k = kbuf[g % _KDEP].astype(jnp.bfloat16)
        if g + _KDEP < n_chunks:
            _k_copy(k_hbm, kbuf, ksem, g + _KDEP).start()
        if g >= _ODEP:
            _o_copy(stag, o_hbm, osem, g - _ODEP).wait()
        stag[g % _ODEP] = jax.lax.dot_general(
            qs, k, (((1,), (1,)), ((), ())),
            preferred_element_type=jnp.float32)
        _o_copy(stag, o_hbm, osem, g).start()

    for g in range(max(0, n_chunks - _ODEP), n_chunks):
        _o_copy(stag, o_hbm, osem, g).wait()


@jax.jit
def kernel(query, key):
    m, d = query.shape
    n = key.shape[0]
    return pl.pallas_call(
        _dist_kernel,
        in_specs=[
            pl.BlockSpec((m, d), lambda: (0, 0)),
            pl.BlockSpec(memory_space=pl.ANY),
        ],
        out_specs=pl.BlockSpec(memory_space=pl.ANY),
        out_shape=jax.ShapeDtypeStruct((m, n), jnp.float32),
        scratch_shapes=[
            pltpu.VMEM((m, d), jnp.bfloat16),
            pltpu.VMEM((_KDEP, _BC, d), jnp.float32),
            pltpu.VMEM((_ODEP, m, _BC), jnp.float32),
            pltpu.SemaphoreType.DMA((_KDEP,)),
            pltpu.SemaphoreType.DMA((_ODEP,)),
        ],
    )(query, key)


# speedup vs baseline: 1.0711x; 1.0711x over previous
import jax
import jax.numpy as jnp
from jax.experimental import pallas as pl
from jax.experimental.pallas import tpu as pltpu

_SCALE = 0.044194173824159216
_TEMPERATURE = 0.2
_C = -_SCALE / _TEMPERATURE

_BN = 8192
_NC = 8
_BC = _BN // _NC
_SLOTS = 4


def _chunk_copy(stag_ref, o_ref, sem, g):
    slot = jax.lax.rem(g, _SLOTS)
    return pltpu.make_async_copy(
        stag_ref.at[slot],
        o_ref.at[:, pl.ds(g * _BC, _BC)],
        sem.at[slot],
    )


def _dist_kernel(q_ref, k_ref, o_ref, qs_ref, stag_ref, sem):
    i = pl.program_id(0)
    nsteps = pl.num_programs(0)

    @pl.when(i == 0)
    def _prep():
        qs_ref[...] = (q_ref[...] * _C).astype(jnp.bfloat16)

    qs = qs_ref[...]
    for c in range(_NC):
        g = i * _NC + c
        if c >= _SLOTS:
            _chunk_copy(stag_ref, o_ref, sem, g - _SLOTS).wait()
        else:
            @pl.when(i >= 1)
            def _reclaim():
                _chunk_copy(stag_ref, o_ref, sem, g - _SLOTS).wait()
        k = k_ref[pl.ds(c * _BC, _BC), :].astype(jnp.bfloat16)
        stag_ref[jax.lax.rem(g, _SLOTS)] = jax.lax.dot_general(
            qs, k, (((1,), (1,)), ((), ())),
            preferred_element_type=jnp.float32)
        _chunk_copy(stag_ref, o_ref, sem, g).start()

    @pl.when(i == nsteps - 1)
    def _drain():
        for c in range(_NC - _SLOTS, _NC):
            _chunk_copy(stag_ref, o_ref, sem, i * _NC + c).wait()


@jax.jit
def kernel(query, key):
    m, d = query.shape
    n = key.shape[0]
    return pl.pallas_call(
        _dist_kernel,
        grid=(n // _BN,),
        in_specs=[
            pl.BlockSpec((m, d), lambda i: (0, 0)),
            pl.BlockSpec((_BN, d), lambda i: (i, 0)),
        ],
        out_specs=pl.BlockSpec(memory_space=pl.ANY),
        out_shape=jax.ShapeDtypeStruct((m, n), jnp.float32),
        scratch_shapes=[
            pltpu.VMEM((m, d), jnp.bfloat16),
            pltpu.VMEM((_SLOTS, m, _BC), jnp.float32),
            pltpu.SemaphoreType.DMA((_SLOTS,)),
        ],
    )(query, key)


# speedup vs baseline: 1.0753x; 1.0039x over previous
import jax
import jax.numpy as jnp
from jax.experimental import pallas as pl
from jax.experimental.pallas import tpu as pltpu

_SCALE = 0.044194173824159216
_TEMPERATURE = 0.2
_C = -_SCALE / _TEMPERATURE

_BN = 8192
_NC = 8
_BC = _BN // _NC
_SLOTS = 5


def _chunk_copy(stag_ref, o_ref, sem, g):
    slot = jax.lax.rem(g, _SLOTS)
    return pltpu.make_async_copy(
        stag_ref.at[slot],
        o_ref.at[:, pl.ds(g * _BC, _BC)],
        sem.at[slot],
    )


def _dist_kernel(q_ref, k_ref, o_ref, qs_ref, stag_ref, sem):
    i = pl.program_id(0)
    nsteps = pl.num_programs(0)

    @pl.when(i == 0)
    def _prep():
        qs_ref[...] = (q_ref[...] * _C).astype(jnp.bfloat16)

    qs = qs_ref[...]
    for c in range(_NC):
        g = i * _NC + c
        if c >= _SLOTS:
            _chunk_copy(stag_ref, o_ref, sem, g - _SLOTS).wait()
        else:
            @pl.when(i >= 1)
            def _reclaim():
                _chunk_copy(stag_ref, o_ref, sem, g - _SLOTS).wait()
        k = k_ref[pl.ds(c * _BC, _BC), :].astype(jnp.bfloat16)
        stag_ref[jax.lax.rem(g, _SLOTS)] = jax.lax.dot_general(
            qs, k, (((1,), (1,)), ((), ())),
            preferred_element_type=jnp.float32)
        _chunk_copy(stag_ref, o_ref, sem, g).start()

    @pl.when(i == nsteps - 1)
    def _drain():
        for c in range(_NC - _SLOTS, _NC):
            _chunk_copy(stag_ref, o_ref, sem, i * _NC + c).wait()


@jax.jit
def kernel(query, key):
    m, d = query.shape
    n = key.shape[0]
    return pl.pallas_call(
        _dist_kernel,
        grid=(n // _BN,),
        in_specs=[
            pl.BlockSpec((m, d), lambda i: (0, 0)),
            pl.BlockSpec((_BN, d), lambda i: (i, 0)),
        ],
        out_specs=pl.BlockSpec(memory_space=pl.ANY),
        out_shape=jax.ShapeDtypeStruct((m, n), jnp.float32),
        scratch_shapes=[
            pltpu.VMEM((m, d), jnp.bfloat16),
            pltpu.VMEM((_SLOTS, m, _BC), jnp.float32),
            pltpu.SemaphoreType.DMA((_SLOTS,)),
        ],
    )(query, key)
